# baseline (speedup 1.0000x reference)
import jax
import jax.numpy as jnp
from jax import lax
from jax.experimental import pallas as pl
from jax.experimental.pallas import tpu as pltpu
from jax.experimental.pallas import tpu_sc as plsc

N = 10000
D = 256
HALF = 128
N_PAD = 10112
ROWS_PER_TILE = N_PAD // 16
E_PAD = 163840
CHUNKS = 80
BATCH = 128
BLK = 1000
C_ROWS = 1280
C_PER_TILE = C_ROWS // 16
REP = 128
GRP = 8
F32 = jnp.float32
_SC_MESH = plsc.VectorSubcoreMesh(core_axis_name="c", subcore_axis_name="s")


def _sc_agg_body(table, src3d, dst3d, zrows, agg_out, src_sl, dst_sl,
                 rows2, agg_s, sem_a, sem_b):
    c = lax.axis_index("c")
    t = lax.axis_index("s")
    r0 = t * ROWS_PER_TILE
    tail = ROWS_PER_TILE - 4 * BATCH

    pltpu.sync_copy(zrows, rows2.at[0])
    for k in range(4):
        pltpu.sync_copy(rows2.at[0], agg_s.at[pl.ds(r0 + k * BATCH, BATCH)])
    pltpu.sync_copy(rows2.at[0, pl.ds(0, tail)],
                    agg_s.at[pl.ds(r0 + 4 * BATCH, tail)])
    plsc.subcore_barrier()

    def gather_start(sref, buf, sem):
        pltpu.async_copy(table.at[sref], rows2.at[buf], sem)

    def gather_wait(sref, buf, sem):
        pltpu.make_async_copy(table.at[sref], rows2.at[buf], sem).wait()

    def scatter(buf, dref):
        pltpu.sync_copy(rows2.at[buf], agg_s.at[dref], add=True)

    def group(g, carry):
        pltpu.sync_copy(src3d.at[c, t, pl.ds(g * GRP, GRP)], src_sl)
        pltpu.sync_copy(dst3d.at[t, pl.ds(g * GRP, GRP)], dst_sl)
        gather_start(src_sl.at[0], 0, sem_a)

        def body(i, carry2):
            r0 = 2 * i
            gather_start(src_sl.at[r0 + 1], 1, sem_b)
            gather_wait(src_sl.at[r0], 0, sem_a)
            scatter(0, dst_sl.at[r0])
            gather_start(src_sl.at[r0 + 2], 0, sem_a)
            gather_wait(src_sl.at[r0 + 1], 1, sem_b)
            scatter(1, dst_sl.at[r0 + 1])
            return carry2

        lax.fori_loop(0, (GRP - 2) // 2, body, 0)
        gather_start(src_sl.at[GRP - 1], 1, sem_b)
        gather_wait(src_sl.at[GRP - 2], 0, sem_a)
        scatter(0, dst_sl.at[GRP - 2])
        gather_wait(src_sl.at[GRP - 1], 1, sem_b)
        scatter(1, dst_sl.at[GRP - 1])
        return carry

    lax.fori_loop(0, CHUNKS // GRP, group, 0)
    plsc.subcore_barrier()

    for k in range(4):
        pltpu.sync_copy(agg_s.at[pl.ds(r0 + k * BATCH, BATCH)], rows2.at[0])
        pltpu.sync_copy(rows2.at[0],
                        agg_out.at[c, pl.ds(r0 + k * BATCH, BATCH)])
    pltpu.sync_copy(agg_s.at[pl.ds(r0 + 4 * BATCH, tail)],
                    rows2.at[0, pl.ds(0, tail)])
    pltpu.sync_copy(rows2.at[0, pl.ds(0, tail)],
                    agg_out.at[c, pl.ds(r0 + 4 * BATCH, tail)])


_sc_agg = pl.kernel(
    _sc_agg_body,
    out_type=jax.ShapeDtypeStruct((2, N_PAD, HALF), F32),
    mesh=_SC_MESH,
    scratch_types=[
        pltpu.VMEM((GRP, BATCH), jnp.int32),
        pltpu.VMEM((GRP, BATCH), jnp.int32),
        pltpu.VMEM((2, BATCH, HALF), F32),
        pltpu.VMEM_SHARED((N_PAD, HALF), F32),
        pltpu.SemaphoreType.DMA,
        pltpu.SemaphoreType.DMA,
    ],
)


def _sc_cnt_body(onehot_tbl, div3d, mod3d, zrows, cnt_out, div_a, mod_a,
                 div_b, mod_b, crows2, cnt_s, sem_a, sem_b):
    c = lax.axis_index("c")
    t = lax.axis_index("s")
    r0 = t * C_PER_TILE
    nk = CHUNKS // 2

    pltpu.sync_copy(zrows.at[pl.ds(0, C_PER_TILE)],
                    cnt_s.at[pl.ds(r0, C_PER_TILE)])
    plsc.subcore_barrier()

    def stage(k, mref, dref):
        jj = 2 * k + c
        pltpu.sync_copy(mod3d.at[t, jj], mref)
        pltpu.sync_copy(div3d.at[t, jj], dref)

    def gather_start(mref, buf, sem):
        pltpu.async_copy(onehot_tbl.at[mref], crows2.at[buf], sem)

    def gather_wait(mref, buf, sem):
        pltpu.make_async_copy(onehot_tbl.at[mref], crows2.at[buf], sem).wait()

    def scatter(buf, dref):
        pltpu.sync_copy(crows2.at[buf], cnt_s.at[dref], add=True)

    stage(0, mod_a, div_a)
    gather_start(mod_a, 0, sem_a)

    def body(i, carry):
        k0 = 2 * i
        stage(k0 + 1, mod_b, div_b)
        gather_start(mod_b, 1, sem_b)
        gather_wait(mod_a, 0, sem_a)
        scatter(0, div_a)
        stage(k0 + 2, mod_a, div_a)
        gather_start(mod_a, 0, sem_a)
        gather_wait(mod_b, 1, sem_b)
        scatter(1, div_b)
        return carry

    lax.fori_loop(0, (nk - 2) // 2, body, 0)
    stage(nk - 1, mod_b, div_b)
    gather_start(mod_b, 1, sem_b)
    gather_wait(mod_a, 0, sem_a)
    scatter(0, div_a)
    gather_wait(mod_b, 1, sem_b)
    scatter(1, div_b)
    plsc.subcore_barrier()

    pltpu.sync_copy(cnt_s.at[pl.ds(r0, C_PER_TILE)],
                    crows2.at[0, pl.ds(0, C_PER_TILE)])
    pltpu.sync_copy(crows2.at[0, pl.ds(0, C_PER_TILE)],
                    cnt_out.at[c, pl.ds(r0, C_PER_TILE)])


_sc_cnt = pl.kernel(
    _sc_cnt_body,
    out_type=jax.ShapeDtypeStruct((2, C_ROWS, HALF), F32),
    mesh=_SC_MESH,
    scratch_types=[
        pltpu.VMEM((BATCH,), jnp.int32),
        pltpu.VMEM((BATCH,), jnp.int32),
        pltpu.VMEM((BATCH,), jnp.int32),
        pltpu.VMEM((BATCH,), jnp.int32),
        pltpu.VMEM((2, BATCH, HALF), F32),
        pltpu.VMEM_SHARED((C_ROWS, HALF), F32),
        pltpu.SemaphoreType.DMA,
        pltpu.SemaphoreType.DMA,
    ],
)


def _xr_body(xs_ref, wr_ref, b_ref, out_ref):
    wr = wr_ref[...]
    out_ref[...] = (jnp.dot(xs_ref[0], wr[:HALF], preferred_element_type=F32)
                    + jnp.dot(xs_ref[1], wr[HALF:], preferred_element_type=F32)
                    + b_ref[...])


def _tc_xr(xsplit, wrT, b2d):
    return pl.pallas_call(
        _xr_body,
        grid=(N // BLK,),
        in_specs=[
            pl.BlockSpec((2, BLK, HALF), lambda i: (0, i, 0)),
            pl.BlockSpec((D, D), lambda i: (0, 0)),
            pl.BlockSpec((1, D), lambda i: (0, 0)),
        ],
        out_specs=pl.BlockSpec((BLK, D), lambda i: (i, 0)),
        out_shape=jax.ShapeDtypeStruct((N, D), F32),
    )(xsplit, wrT, b2d)


def _combine_body(agg_ref, cnt_ref, xr_ref, wl_ref, outs_ref, outf_ref):
    cnt = jnp.sum(cnt_ref[0] + cnt_ref[1], axis=1, keepdims=True)
    inv = 1.0 / jnp.maximum(cnt, 1.0)
    mL = agg_ref[0] * inv
    mR = agg_ref[1] * inv
    wl = wl_ref[...]
    acc = (jnp.dot(mL, wl[:HALF], preferred_element_type=F32)
           + jnp.dot(mR, wl[HALF:], preferred_element_type=F32)
           + xr_ref[...])
    norm = jnp.sqrt(jnp.sum(acc * acc, axis=1, keepdims=True))
    acc = acc / jnp.maximum(norm, 1e-12)
    acc = jnp.maximum(acc, 0.0)
    outf_ref[...] = acc
    outs_ref[0] = acc[:, :HALF]
    outs_ref[1] = acc[:, HALF:]


def _tc_combine(agg2x, cnt16, xr, wlT):
    return pl.pallas_call(
        _combine_body,
        grid=(N // BLK,),
        in_specs=[
            pl.BlockSpec((2, BLK, HALF), lambda i: (0, i, 0)),
            pl.BlockSpec((2, BLK, 16), lambda i: (0, i, 0)),
            pl.BlockSpec((BLK, D), lambda i: (i, 0)),
            pl.BlockSpec((D, D), lambda i: (0, 0)),
        ],
        out_specs=[
            pl.BlockSpec((2, BLK, HALF), lambda i: (0, i, 0)),
            pl.BlockSpec((BLK, D), lambda i: (i, 0)),
        ],
        out_shape=[
            jax.ShapeDtypeStruct((2, N, HALF), F32),
            jax.ShapeDtypeStruct((N, D), F32),
        ],
    )(agg2x, cnt16, xr, wlT)


def kernel(x, edge_index, W1l, b1l, W1r, W2l, b2l, W2r):
    ei = edge_index.astype(jnp.int32)
    npad = E_PAD - ei.shape[1]
    src = jnp.concatenate([ei[0], jnp.zeros((npad,), jnp.int32)])
    dst = jnp.concatenate([ei[1], jnp.full((npad,), N, jnp.int32)])
    dst3d = dst.reshape(16, CHUNKS, BATCH)
    src3d = jnp.stack([src, src + N]).reshape(2, 16, CHUNKS, BATCH)
    div3d = (dst // 8).reshape(16, CHUNKS, BATCH)
    lane_spread = jnp.arange(E_PAD, dtype=jnp.int32) % REP
    mod3d = (dst % 8 + 8 * lane_spread).reshape(16, CHUNKS, BATCH)

    xsplit = jnp.stack([x[:, :HALF], x[:, HALF:]])
    xflat = xsplit.reshape(2 * N, HALF)

    zrows = jnp.zeros((BATCH, HALF), F32)
    onehot_row = jnp.zeros((8, HALF), F32)
    onehot_row = onehot_row.at[jnp.arange(8), jnp.arange(8) * 16].set(1.0)
    onehot_tbl = jnp.tile(onehot_row, (REP, 1))

    w1lT, w1rT = W1l.T, W1r.T
    w2lT, w2rT = W2l.T, W2r.T
    b1 = b1l.reshape(1, D)
    b2 = b2l.reshape(1, D)

    cnt_pk = _sc_cnt(onehot_tbl, div3d, mod3d, zrows)
    cnt16 = cnt_pk.reshape(2, 8 * C_ROWS, 16)
    agg1 = _sc_agg(xflat, src3d, dst3d, zrows)
    xr1 = _tc_xr(xsplit, w1rT, b1)
    hs, _ = _tc_combine(agg1, cnt16, xr1, w1lT)
    hflat = hs.reshape(2 * N, HALF)
    agg2 = _sc_agg(hflat, src3d, dst3d, zrows)
    xr2 = _tc_xr(hs, w2rT, b2)
    _, out = _tc_combine(agg2, cnt16, xr2, w2lT)
    return out


# speedup vs baseline: 1.0796x; 1.0796x over previous
import jax
import jax.numpy as jnp
from jax import lax
from jax.experimental import pallas as pl
from jax.experimental.pallas import tpu as pltpu
from jax.experimental.pallas import tpu_sc as plsc

N = 10000
D = 256
HALF = 128
N_PAD = 10112
ROWS_PER_TILE = N_PAD // 16
E_PAD = 163840
CHUNKS = 80
BATCH = 128
BLK = 1000
C_ROWS = 1280
C_PER_TILE = C_ROWS // 16
REP = 128
GRP = 8
F32 = jnp.float32
_SC_MESH = plsc.VectorSubcoreMesh(core_axis_name="c", subcore_axis_name="s")


def _sc_agg_body(table, src3d, dst3d, zrows, agg_out, src_sl, dst_sl,
                 rows2, agg_s, sem_a, sem_b):
    c = lax.axis_index("c")
    t = lax.axis_index("s")
    r0 = t * ROWS_PER_TILE
    tail = ROWS_PER_TILE - 4 * BATCH

    pltpu.sync_copy(zrows, rows2.at[0])
    for k in range(4):
        pltpu.sync_copy(rows2.at[0], agg_s.at[pl.ds(r0 + k * BATCH, BATCH)])
    pltpu.sync_copy(rows2.at[0, pl.ds(0, tail)],
                    agg_s.at[pl.ds(r0 + 4 * BATCH, tail)])
    plsc.subcore_barrier()

    def gather_start(sref, buf, sem):
        pltpu.async_copy(table.at[sref], rows2.at[buf], sem)

    def gather_wait(sref, buf, sem):
        pltpu.make_async_copy(table.at[sref], rows2.at[buf], sem).wait()

    def scatter(buf, dref):
        pltpu.sync_copy(rows2.at[buf], agg_s.at[dref], add=True)

    def group(g, carry):
        pltpu.sync_copy(src3d.at[c, t, pl.ds(g * GRP, GRP)], src_sl)
        pltpu.sync_copy(dst3d.at[t, pl.ds(g * GRP, GRP)], dst_sl)
        gather_start(src_sl.at[0], 0, sem_a)

        def body(i, carry2):
            r0 = 2 * i
            gather_start(src_sl.at[r0 + 1], 1, sem_b)
            gather_wait(src_sl.at[r0], 0, sem_a)
            scatter(0, dst_sl.at[r0])
            gather_start(src_sl.at[r0 + 2], 0, sem_a)
            gather_wait(src_sl.at[r0 + 1], 1, sem_b)
            scatter(1, dst_sl.at[r0 + 1])
            return carry2

        lax.fori_loop(0, (GRP - 2) // 2, body, 0)
        gather_start(src_sl.at[GRP - 1], 1, sem_b)
        gather_wait(src_sl.at[GRP - 2], 0, sem_a)
        scatter(0, dst_sl.at[GRP - 2])
        gather_wait(src_sl.at[GRP - 1], 1, sem_b)
        scatter(1, dst_sl.at[GRP - 1])
        return carry

    lax.fori_loop(0, CHUNKS // GRP, group, 0)
    plsc.subcore_barrier()

    for k in range(4):
        pltpu.sync_copy(agg_s.at[pl.ds(r0 + k * BATCH, BATCH)], rows2.at[0])
        pltpu.sync_copy(rows2.at[0],
                        agg_out.at[c, pl.ds(r0 + k * BATCH, BATCH)])
    pltpu.sync_copy(agg_s.at[pl.ds(r0 + 4 * BATCH, tail)],
                    rows2.at[0, pl.ds(0, tail)])
    pltpu.sync_copy(rows2.at[0, pl.ds(0, tail)],
                    agg_out.at[c, pl.ds(r0 + 4 * BATCH, tail)])


_sc_agg = pl.kernel(
    _sc_agg_body,
    out_type=jax.ShapeDtypeStruct((2, N_PAD, HALF), F32),
    mesh=_SC_MESH,
    scratch_types=[
        pltpu.VMEM((GRP, BATCH), jnp.int32),
        pltpu.VMEM((GRP, BATCH), jnp.int32),
        pltpu.VMEM((2, BATCH, HALF), F32),
        pltpu.VMEM_SHARED((N_PAD, HALF), F32),
        pltpu.SemaphoreType.DMA,
        pltpu.SemaphoreType.DMA,
    ],
)


def _sc_cnt_body(onehot_tbl, div3d, mod3d, zrows, cnt_out, div_a, mod_a,
                 div_b, mod_b, crows2, cnt_s, sem_a, sem_b):
    c = lax.axis_index("c")
    t = lax.axis_index("s")
    r0 = t * C_PER_TILE
    nk = CHUNKS // 2

    pltpu.sync_copy(zrows.at[pl.ds(0, C_PER_TILE)],
                    cnt_s.at[pl.ds(r0, C_PER_TILE)])
    plsc.subcore_barrier()

    def stage(k, mref, dref):
        jj = 2 * k + c
        pltpu.sync_copy(mod3d.at[t, jj], mref)
        pltpu.sync_copy(div3d.at[t, jj], dref)

    def gather_start(mref, buf, sem):
        pltpu.async_copy(onehot_tbl.at[mref], crows2.at[buf], sem)

    def gather_wait(mref, buf, sem):
        pltpu.make_async_copy(onehot_tbl.at[mref], crows2.at[buf], sem).wait()

    def scatter(buf, dref):
        pltpu.sync_copy(crows2.at[buf], cnt_s.at[dref], add=True)

    stage(0, mod_a, div_a)
    gather_start(mod_a, 0, sem_a)

    def body(i, carry):
        k0 = 2 * i
        stage(k0 + 1, mod_b, div_b)
        gather_start(mod_b, 1, sem_b)
        gather_wait(mod_a, 0, sem_a)
        scatter(0, div_a)
        stage(k0 + 2, mod_a, div_a)
        gather_start(mod_a, 0, sem_a)
        gather_wait(mod_b, 1, sem_b)
        scatter(1, div_b)
        return carry

    lax.fori_loop(0, (nk - 2) // 2, body, 0)
    stage(nk - 1, mod_b, div_b)
    gather_start(mod_b, 1, sem_b)
    gather_wait(mod_a, 0, sem_a)
    scatter(0, div_a)
    gather_wait(mod_b, 1, sem_b)
    scatter(1, div_b)
    plsc.subcore_barrier()

    pltpu.sync_copy(cnt_s.at[pl.ds(r0, C_PER_TILE)],
                    crows2.at[0, pl.ds(0, C_PER_TILE)])
    pltpu.sync_copy(crows2.at[0, pl.ds(0, C_PER_TILE)],
                    cnt_out.at[c, pl.ds(r0, C_PER_TILE)])


_sc_cnt = pl.kernel(
    _sc_cnt_body,
    out_type=jax.ShapeDtypeStruct((2, C_ROWS, HALF), F32),
    mesh=_SC_MESH,
    scratch_types=[
        pltpu.VMEM((BATCH,), jnp.int32),
        pltpu.VMEM((BATCH,), jnp.int32),
        pltpu.VMEM((BATCH,), jnp.int32),
        pltpu.VMEM((BATCH,), jnp.int32),
        pltpu.VMEM((2, BATCH, HALF), F32),
        pltpu.VMEM_SHARED((C_ROWS, HALF), F32),
        pltpu.SemaphoreType.DMA,
        pltpu.SemaphoreType.DMA,
    ],
)


def _combine_body(agg_ref, cnt_ref, xs_ref, wl_ref, wr_ref, b_ref,
                  outs_ref, outf_ref):
    cnt = jnp.sum(cnt_ref[0] + cnt_ref[1], axis=1, keepdims=True)
    inv = 1.0 / jnp.maximum(cnt, 1.0)
    mL = agg_ref[0] * inv
    mR = agg_ref[1] * inv
    wl = wl_ref[...]
    wr = wr_ref[...]
    acc = (jnp.dot(mL, wl[:HALF], preferred_element_type=F32)
           + jnp.dot(mR, wl[HALF:], preferred_element_type=F32)
           + jnp.dot(xs_ref[0], wr[:HALF], preferred_element_type=F32)
           + jnp.dot(xs_ref[1], wr[HALF:], preferred_element_type=F32)
           + b_ref[...])
    norm = jnp.sqrt(jnp.sum(acc * acc, axis=1, keepdims=True))
    acc = acc / jnp.maximum(norm, 1e-12)
    acc = jnp.maximum(acc, 0.0)
    outf_ref[...] = acc
    outs_ref[0] = acc[:, :HALF]
    outs_ref[1] = acc[:, HALF:]


def _tc_combine(agg2x, cnt16, xsplit, wlT, wrT, b2d):
    return pl.pallas_call(
        _combine_body,
        grid=(N // BLK,),
        in_specs=[
            pl.BlockSpec((2, BLK, HALF), lambda i: (0, i, 0)),
            pl.BlockSpec((2, BLK, 16), lambda i: (0, i, 0)),
            pl.BlockSpec((2, BLK, HALF), lambda i: (0, i, 0)),
            pl.BlockSpec((D, D), lambda i: (0, 0)),
            pl.BlockSpec((D, D), lambda i: (0, 0)),
            pl.BlockSpec((1, D), lambda i: (0, 0)),
        ],
        out_specs=[
            pl.BlockSpec((2, BLK, HALF), lambda i: (0, i, 0)),
            pl.BlockSpec((BLK, D), lambda i: (i, 0)),
        ],
        out_shape=[
            jax.ShapeDtypeStruct((2, N, HALF), F32),
            jax.ShapeDtypeStruct((N, D), F32),
        ],
    )(agg2x, cnt16, xsplit, wlT, wrT, b2d)


def kernel(x, edge_index, W1l, b1l, W1r, W2l, b2l, W2r):
    ei = edge_index.astype(jnp.int32)
    npad = E_PAD - ei.shape[1]
    src = jnp.concatenate([ei[0], jnp.zeros((npad,), jnp.int32)])
    dst = jnp.concatenate([ei[1], jnp.full((npad,), N, jnp.int32)])
    dst3d = dst.reshape(16, CHUNKS, BATCH)
    src3d = jnp.stack([src, src + N]).reshape(2, 16, CHUNKS, BATCH)
    div3d = (dst // 8).reshape(16, CHUNKS, BATCH)
    lane_spread = jnp.arange(E_PAD, dtype=jnp.int32) % REP
    mod3d = (dst % 8 + 8 * lane_spread).reshape(16, CHUNKS, BATCH)

    xsplit = jnp.stack([x[:, :HALF], x[:, HALF:]])
    xflat = xsplit.reshape(2 * N, HALF)

    zrows = jnp.zeros((BATCH, HALF), F32)
    onehot_row = jnp.zeros((8, HALF), F32)
    onehot_row = onehot_row.at[jnp.arange(8), jnp.arange(8) * 16].set(1.0)
    onehot_tbl = jnp.tile(onehot_row, (REP, 1))

    w1lT, w1rT = W1l.T, W1r.T
    w2lT, w2rT = W2l.T, W2r.T
    b1 = b1l.reshape(1, D)
    b2 = b2l.reshape(1, D)

    cnt_pk = _sc_cnt(onehot_tbl, div3d, mod3d, zrows)
    cnt16 = cnt_pk.reshape(2, 8 * C_ROWS, 16)
    agg1 = _sc_agg(xflat, src3d, dst3d, zrows)
    hs, _ = _tc_combine(agg1, cnt16, xsplit, w1lT, w1rT, b1)
    hflat = hs.reshape(2 * N, HALF)
    agg2 = _sc_agg(hflat, src3d, dst3d, zrows)
    _, out = _tc_combine(agg2, cnt16, hs, w2lT, w2rT, b2)
    return out
